# baseline (speedup 1.0000x reference)
import functools

import jax
import jax.numpy as jnp
from jax import lax
from jax.experimental import pallas as pl
from jax.experimental.pallas import tpu as pltpu
from jax.experimental.pallas import tpu_sc as plsc

N = 10000
E = 320000
D_IN = 128
EMB = 128
HID = 128
OUT = 64
L = 3

NC = 2
NS = 16
NW = NC * NS
EPW = E // NW
CH = 50
NCHUNK = EPW // CH
SEG = 5
CPS = NCHUNK // SEG
NSLOT = 4
RPB = 624
TAIL = N - NS * RPB

ROWS_B = 1000
GRID = N // ROWS_B


def _ln_rows(t, s, b):
    mu = jnp.mean(t, axis=-1, keepdims=True)
    var = jnp.mean((t - mu) ** 2, axis=-1, keepdims=True)
    return (t - mu) * lax.rsqrt(var + 1e-5) * s + b


def _in_msg_body(x_ref, win_ref, bin_ref, deg_ref, w1_ref, b1_ref,
                 h_ref, p_ref):
    h = jax.nn.relu(
        jnp.dot(x_ref[...], win_ref[...], preferred_element_type=jnp.float32)
        + bin_ref[...]
    )
    h_ref[...] = h
    inv = lax.rsqrt(jnp.maximum(deg_ref[...] + 1.0, 1.0))
    m = jax.nn.relu(
        jnp.dot(h, w1_ref[...], preferred_element_type=jnp.float32)
        + b1_ref[...]
    )
    p_ref[...] = m * inv


def _upd_msg_body(q0_ref, q1_ref, p_ref, h_ref, deg_ref, w_ref, b_ref,
                  s_ref, lb_ref, w1_ref, b1_ref, h_ref_o, p_ref_o):
    inv = lax.rsqrt(jnp.maximum(deg_ref[...] + 1.0, 1.0))
    agg = (q0_ref[...] + q1_ref[...] + p_ref[...]) * inv
    t = jnp.dot(agg, w_ref[...], preferred_element_type=jnp.float32) + b_ref[...]
    hn = _ln_rows(t, s_ref[...], lb_ref[...]) + h_ref[...]
    h_ref_o[...] = hn
    m = jax.nn.relu(
        jnp.dot(hn, w1_ref[...], preferred_element_type=jnp.float32)
        + b1_ref[...]
    )
    p_ref_o[...] = m * inv


def _upd_out_body(q0_ref, q1_ref, p_ref, h_ref, deg_ref, w_ref, b_ref,
                  s_ref, lb_ref, os_ref, ob_ref, wo_ref, bo_ref, o_ref):
    inv = lax.rsqrt(jnp.maximum(deg_ref[...] + 1.0, 1.0))
    agg = (q0_ref[...] + q1_ref[...] + p_ref[...]) * inv
    t = jnp.dot(agg, w_ref[...], preferred_element_type=jnp.float32) + b_ref[...]
    hn = _ln_rows(t, s_ref[...], lb_ref[...]) + h_ref[...]
    t2 = _ln_rows(hn, os_ref[...], ob_ref[...])
    o_ref[...] = (
        jnp.dot(t2, wo_ref[...], preferred_element_type=jnp.float32)
        + bo_ref[...]
    )


def _row_spec(cols):
    return pl.BlockSpec((ROWS_B, cols), lambda i: (i, 0))


def _rep_spec(r, c):
    return pl.BlockSpec((r, c), lambda i: (0, 0))


_in_msg_call = pl.pallas_call(
    _in_msg_body,
    grid=(GRID,),
    in_specs=[
        _row_spec(D_IN),
        _rep_spec(D_IN, EMB),
        _rep_spec(1, EMB),
        _row_spec(1),
        _rep_spec(EMB, HID),
        _rep_spec(1, HID),
    ],
    out_specs=[_row_spec(EMB), _row_spec(HID)],
    out_shape=[
        jax.ShapeDtypeStruct((N, EMB), jnp.float32),
        jax.ShapeDtypeStruct((N, HID), jnp.float32),
    ],
)

_upd_msg_call = pl.pallas_call(
    _upd_msg_body,
    grid=(GRID,),
    in_specs=[
        _row_spec(HID),
        _row_spec(HID),
        _row_spec(HID),
        _row_spec(EMB),
        _row_spec(1),
        _rep_spec(HID, EMB),
        _rep_spec(1, EMB),
        _rep_spec(1, EMB),
        _rep_spec(1, EMB),
        _rep_spec(EMB, HID),
        _rep_spec(1, HID),
    ],
    out_specs=[_row_spec(EMB), _row_spec(HID)],
    out_shape=[
        jax.ShapeDtypeStruct((N, EMB), jnp.float32),
        jax.ShapeDtypeStruct((N, HID), jnp.float32),
    ],
)

_upd_out_call = pl.pallas_call(
    _upd_out_body,
    grid=(GRID,),
    in_specs=[
        _row_spec(HID),
        _row_spec(HID),
        _row_spec(HID),
        _row_spec(EMB),
        _row_spec(1),
        _rep_spec(HID, EMB),
        _rep_spec(1, EMB),
        _rep_spec(1, EMB),
        _rep_spec(1, EMB),
        _rep_spec(1, EMB),
        _rep_spec(1, EMB),
        _rep_spec(EMB, OUT),
        _rep_spec(1, OUT),
    ],
    out_specs=_row_spec(OUT),
    out_shape=jax.ShapeDtypeStruct((N, OUT), jnp.float32),
)


def _sc_body(p_hbm, src_hbm, dst_hbm, zeros_hbm, out_hbm,
             src_a, dst_a, src_b, dst_b, rows0, rows1, rows2, rows3, acc,
             lsem, gsem, ssem):
    rows = (rows0, rows1, rows2, rows3)
    slabs = ((src_a, dst_a), (src_b, dst_b))
    c = lax.axis_index("c")
    s = lax.axis_index("s")
    wid = c * NS + s

    def load_slab(seg, t):
        pltpu.async_copy(src_hbm.at[wid, seg], slabs[t][0], lsem.at[t])
        pltpu.async_copy(dst_hbm.at[wid, seg], slabs[t][1], lsem.at[t])

    def wait_slab(t):
        pltpu.make_async_copy(src_hbm.at[wid, 0], slabs[t][0],
                              lsem.at[t]).wait()
        pltpu.make_async_copy(dst_hbm.at[wid, 0], slabs[t][1],
                              lsem.at[t]).wait()

    load_slab(0, 0)
    load_slab(1, 1)
    pltpu.sync_copy(zeros_hbm.at[pl.ds(0, RPB)], acc.at[pl.ds(s * RPB, RPB)])

    @pl.when(s == 0)
    def _zero_tail():
        pltpu.sync_copy(zeros_hbm.at[pl.ds(0, TAIL)],
                        acc.at[pl.ds(NS * RPB, TAIL)])

    plsc.subcore_barrier()

    def issue_gather(sv, cc, b):
        pltpu.async_copy(p_hbm.at[sv.at[cc]], rows[b], gsem.at[b])

    def wait_gather(sv, cc, b):
        pltpu.make_async_copy(p_hbm.at[sv.at[cc]], rows[b],
                              gsem.at[b]).wait()

    def issue_scatter(dv, cc, b):
        pltpu.async_copy(rows[b], acc.at[dv.at[cc]], ssem.at[b], add=True)

    def wait_scatter(dv, cc, b):
        pltpu.make_async_copy(rows[b], acc.at[dv.at[cc]],
                              ssem.at[b]).wait()

    def step(sv, dv, cc, b, do_swait, do_gather):
        wait_gather(sv, cc, b)
        if do_swait:
            wait_scatter(dv, cc - 2, (b + 2) % NSLOT)
        if do_gather:
            issue_gather(sv, cc + 2, (b + 2) % NSLOT)
        issue_scatter(dv, cc, b)

    for seg in range(SEG):
        t = seg % 2
        sv, dv = slabs[t]
        wait_slab(t)
        issue_gather(sv, 0, 0)
        issue_gather(sv, 1, 1)
        step(sv, dv, 0, 0, do_swait=False, do_gather=True)
        step(sv, dv, 1, 1, do_swait=False, do_gather=True)

        def trip(j, carry, sv=sv, dv=dv):
            cb = NSLOT * j + 2
            step(sv, dv, cb, 2, True, True)
            step(sv, dv, cb + 1, 3, True, True)
            step(sv, dv, cb + 2, 0, True, True)
            step(sv, dv, cb + 3, 1, True, True)
            return carry

        lax.fori_loop(0, (CPS - 4) // NSLOT, trip, 0)
        step(sv, dv, CPS - 2, (CPS - 2) % NSLOT, True, False)
        step(sv, dv, CPS - 1, (CPS - 1) % NSLOT, True, False)
        wait_scatter(dv, CPS - 2, (CPS - 2) % NSLOT)
        wait_scatter(dv, CPS - 1, (CPS - 1) % NSLOT)
        if seg + 2 < SEG:
            load_slab(seg + 2, t)

    plsc.subcore_barrier()
    pltpu.sync_copy(acc.at[pl.ds(s * RPB, RPB)],
                    out_hbm.at[c].at[pl.ds(s * RPB, RPB)])

    @pl.when(s == 0)
    def _drain_tail():
        pltpu.sync_copy(acc.at[pl.ds(NS * RPB, TAIL)],
                        out_hbm.at[c].at[pl.ds(NS * RPB, TAIL)])


@functools.lru_cache(maxsize=None)
def _make_sc_call():
    return functools.partial(
        pl.kernel,
        out_type=jax.ShapeDtypeStruct((NC, N, EMB), jnp.float32),
        mesh=plsc.VectorSubcoreMesh(core_axis_name="c", subcore_axis_name="s",
                                    num_cores=NC, num_subcores=NS),
        scratch_types=[
            pltpu.VMEM((CPS, CH), jnp.int32),
            pltpu.VMEM((CPS, CH), jnp.int32),
            pltpu.VMEM((CPS, CH), jnp.int32),
            pltpu.VMEM((CPS, CH), jnp.int32),
            pltpu.VMEM((CH, EMB), jnp.float32),
            pltpu.VMEM((CH, EMB), jnp.float32),
            pltpu.VMEM((CH, EMB), jnp.float32),
            pltpu.VMEM((CH, EMB), jnp.float32),
            pltpu.VMEM_SHARED((N, EMB), jnp.float32),
            pltpu.SemaphoreType.DMA((2,)),
            pltpu.SemaphoreType.DMA((NSLOT,)),
            pltpu.SemaphoreType.DMA((NSLOT,)),
        ],
    )(_sc_body)


def kernel(x, degree, W_in, b_in, W1, b1, W2, b2, ln_s, ln_b,
           out_ln_s, out_ln_b, W_out, b_out, edge_index):
    deg = degree.reshape(N, 1)
    src = edge_index[0].reshape(NW, SEG, CPS, CH)
    dst = edge_index[1].reshape(NW, SEG, CPS, CH)
    zeros = jnp.zeros((RPB, EMB), jnp.float32)

    h, p = _in_msg_call(x, W_in, b_in.reshape(1, EMB), deg,
                        W1[0], b1[0].reshape(1, HID))
    for l in range(L - 1):
        q = _make_sc_call()(p, src, dst, zeros)
        h, p = _upd_msg_call(q[0], q[1], p, h, deg, W2[l],
                             b2[l].reshape(1, EMB), ln_s[l].reshape(1, EMB),
                             ln_b[l].reshape(1, EMB), W1[l + 1],
                             b1[l + 1].reshape(1, HID))
    q = _make_sc_call()(p, src, dst, zeros)
    return _upd_out_call(q[0], q[1], p, h, deg, W2[L - 1],
                         b2[L - 1].reshape(1, EMB),
                         ln_s[L - 1].reshape(1, EMB),
                         ln_b[L - 1].reshape(1, EMB),
                         out_ln_s.reshape(1, EMB), out_ln_b.reshape(1, EMB),
                         W_out, b_out.reshape(1, OUT))


# speedup vs baseline: 1.1824x; 1.1824x over previous
import functools

import jax
import jax.numpy as jnp
from jax import lax
from jax.experimental import pallas as pl
from jax.experimental.pallas import tpu as pltpu
from jax.experimental.pallas import tpu_sc as plsc

N = 10000
E = 320000
D_IN = 128
EMB = 128
HID = 128
OUT = 64
L = 3

NC = 2
NS = 16
NW = NC * NS
EPW = E // NW
CH = 80
NCHUNK = EPW // CH
SEG = 5
CPS = NCHUNK // SEG
NSLOT = 3
RPB = 624
TAIL = N - NS * RPB

ROWS_B = 1000
GRID = N // ROWS_B


def _ln_rows(t, s, b):
    mu = jnp.mean(t, axis=-1, keepdims=True)
    var = jnp.mean((t - mu) ** 2, axis=-1, keepdims=True)
    return (t - mu) * lax.rsqrt(var + 1e-5) * s + b


def _in_msg_body(x_ref, win_ref, bin_ref, deg_ref, w1_ref, b1_ref,
                 h_ref, p_ref):
    h = jax.nn.relu(
        jnp.dot(x_ref[...], win_ref[...], preferred_element_type=jnp.float32)
        + bin_ref[...]
    )
    h_ref[...] = h
    inv = lax.rsqrt(jnp.maximum(deg_ref[...] + 1.0, 1.0))
    m = jax.nn.relu(
        jnp.dot(h, w1_ref[...], preferred_element_type=jnp.float32)
        + b1_ref[...]
    )
    p_ref[...] = m * inv


def _upd_msg_body(q0_ref, q1_ref, p_ref, h_ref, deg_ref, w_ref, b_ref,
                  s_ref, lb_ref, w1_ref, b1_ref, h_ref_o, p_ref_o):
    inv = lax.rsqrt(jnp.maximum(deg_ref[...] + 1.0, 1.0))
    agg = (q0_ref[...] + q1_ref[...] + p_ref[...]) * inv
    t = jnp.dot(agg, w_ref[...], preferred_element_type=jnp.float32) + b_ref[...]
    hn = _ln_rows(t, s_ref[...], lb_ref[...]) + h_ref[...]
    h_ref_o[...] = hn
    m = jax.nn.relu(
        jnp.dot(hn, w1_ref[...], preferred_element_type=jnp.float32)
        + b1_ref[...]
    )
    p_ref_o[...] = m * inv


def _upd_out_body(q0_ref, q1_ref, p_ref, h_ref, deg_ref, w_ref, b_ref,
                  s_ref, lb_ref, os_ref, ob_ref, wo_ref, bo_ref, o_ref):
    inv = lax.rsqrt(jnp.maximum(deg_ref[...] + 1.0, 1.0))
    agg = (q0_ref[...] + q1_ref[...] + p_ref[...]) * inv
    t = jnp.dot(agg, w_ref[...], preferred_element_type=jnp.float32) + b_ref[...]
    hn = _ln_rows(t, s_ref[...], lb_ref[...]) + h_ref[...]
    t2 = _ln_rows(hn, os_ref[...], ob_ref[...])
    o_ref[...] = (
        jnp.dot(t2, wo_ref[...], preferred_element_type=jnp.float32)
        + bo_ref[...]
    )


def _row_spec(cols):
    return pl.BlockSpec((ROWS_B, cols), lambda i: (i, 0))


def _rep_spec(r, c):
    return pl.BlockSpec((r, c), lambda i: (0, 0))


_in_msg_call = pl.pallas_call(
    _in_msg_body,
    grid=(GRID,),
    in_specs=[
        _row_spec(D_IN),
        _rep_spec(D_IN, EMB),
        _rep_spec(1, EMB),
        _row_spec(1),
        _rep_spec(EMB, HID),
        _rep_spec(1, HID),
    ],
    out_specs=[_row_spec(EMB), _row_spec(HID)],
    out_shape=[
        jax.ShapeDtypeStruct((N, EMB), jnp.float32),
        jax.ShapeDtypeStruct((N, HID), jnp.float32),
    ],
)

_upd_msg_call = pl.pallas_call(
    _upd_msg_body,
    grid=(GRID,),
    in_specs=[
        _row_spec(HID),
        _row_spec(HID),
        _row_spec(HID),
        _row_spec(EMB),
        _row_spec(1),
        _rep_spec(HID, EMB),
        _rep_spec(1, EMB),
        _rep_spec(1, EMB),
        _rep_spec(1, EMB),
        _rep_spec(EMB, HID),
        _rep_spec(1, HID),
    ],
    out_specs=[_row_spec(EMB), _row_spec(HID)],
    out_shape=[
        jax.ShapeDtypeStruct((N, EMB), jnp.float32),
        jax.ShapeDtypeStruct((N, HID), jnp.float32),
    ],
)

_upd_out_call = pl.pallas_call(
    _upd_out_body,
    grid=(GRID,),
    in_specs=[
        _row_spec(HID),
        _row_spec(HID),
        _row_spec(HID),
        _row_spec(EMB),
        _row_spec(1),
        _rep_spec(HID, EMB),
        _rep_spec(1, EMB),
        _rep_spec(1, EMB),
        _rep_spec(1, EMB),
        _rep_spec(1, EMB),
        _rep_spec(1, EMB),
        _rep_spec(EMB, OUT),
        _rep_spec(1, OUT),
    ],
    out_specs=_row_spec(OUT),
    out_shape=jax.ShapeDtypeStruct((N, OUT), jnp.float32),
)


def _sc_body(p_hbm, src_hbm, dst_hbm, zeros_hbm, out_hbm,
             src_a, dst_a, src_b, dst_b, rows0, rows1, rows2, acc,
             lsem, gsem, ssem):
    rows = (rows0, rows1, rows2)
    slabs = ((src_a, dst_a), (src_b, dst_b))
    c = lax.axis_index("c")
    s = lax.axis_index("s")
    wid = c * NS + s

    def load_slab(seg, t):
        pltpu.async_copy(src_hbm.at[wid, seg], slabs[t][0], lsem.at[t])
        pltpu.async_copy(dst_hbm.at[wid, seg], slabs[t][1], lsem.at[t])

    def wait_slab(t):
        pltpu.make_async_copy(src_hbm.at[wid, 0], slabs[t][0],
                              lsem.at[t]).wait()
        pltpu.make_async_copy(dst_hbm.at[wid, 0], slabs[t][1],
                              lsem.at[t]).wait()

    load_slab(0, 0)
    load_slab(1, 1)
    pltpu.sync_copy(zeros_hbm.at[pl.ds(0, RPB)], acc.at[pl.ds(s * RPB, RPB)])

    @pl.when(s == 0)
    def _zero_tail():
        pltpu.sync_copy(zeros_hbm.at[pl.ds(0, TAIL)],
                        acc.at[pl.ds(NS * RPB, TAIL)])

    plsc.subcore_barrier()

    def issue_gather(sv, cc, b):
        pltpu.async_copy(p_hbm.at[sv.at[cc]], rows[b], gsem.at[b])

    def wait_gather(sv, cc, b):
        pltpu.make_async_copy(p_hbm.at[sv.at[cc]], rows[b],
                              gsem.at[b]).wait()

    def issue_scatter(dv, cc, b):
        pltpu.async_copy(rows[b], acc.at[dv.at[cc]], ssem.at[b], add=True)

    def wait_scatter(dv, cc, b):
        pltpu.make_async_copy(rows[b], acc.at[dv.at[cc]],
                              ssem.at[b]).wait()

    def step(sv, dv, cc, b, do_swait, do_gather):
        wait_gather(sv, cc, b)
        issue_scatter(dv, cc, b)
        if do_swait:
            wait_scatter(dv, cc - 1, (b + 2) % NSLOT)
        if do_gather:
            issue_gather(sv, cc + 2, (b + 2) % NSLOT)

    for seg in range(SEG):
        t = seg % 2
        sv, dv = slabs[t]
        wait_slab(t)
        issue_gather(sv, 0, 0)
        issue_gather(sv, 1, 1)
        step(sv, dv, 0, 0, do_swait=False, do_gather=True)
        step(sv, dv, 1, 1, do_swait=True, do_gather=True)

        def trip(j, carry, sv=sv, dv=dv):
            cb = NSLOT * j + 2
            step(sv, dv, cb, 2, True, True)
            step(sv, dv, cb + 1, 0, True, True)
            step(sv, dv, cb + 2, 1, True, True)
            return carry

        lax.fori_loop(0, (CPS - 4) // NSLOT, trip, 0)
        step(sv, dv, CPS - 2, (CPS - 2) % NSLOT, True, False)
        step(sv, dv, CPS - 1, (CPS - 1) % NSLOT, True, False)
        wait_scatter(dv, CPS - 1, (CPS - 1) % NSLOT)
        if seg + 2 < SEG:
            load_slab(seg + 2, t)

    plsc.subcore_barrier()
    pltpu.sync_copy(acc.at[pl.ds(s * RPB, RPB)],
                    out_hbm.at[c].at[pl.ds(s * RPB, RPB)])

    @pl.when(s == 0)
    def _drain_tail():
        pltpu.sync_copy(acc.at[pl.ds(NS * RPB, TAIL)],
                        out_hbm.at[c].at[pl.ds(NS * RPB, TAIL)])


@functools.lru_cache(maxsize=None)
def _make_sc_call():
    return functools.partial(
        pl.kernel,
        out_type=jax.ShapeDtypeStruct((NC, N, EMB), jnp.float32),
        mesh=plsc.VectorSubcoreMesh(core_axis_name="c", subcore_axis_name="s",
                                    num_cores=NC, num_subcores=NS),
        scratch_types=[
            pltpu.VMEM((CPS, CH), jnp.int32),
            pltpu.VMEM((CPS, CH), jnp.int32),
            pltpu.VMEM((CPS, CH), jnp.int32),
            pltpu.VMEM((CPS, CH), jnp.int32),
            pltpu.VMEM((CH, EMB), jnp.float32),
            pltpu.VMEM((CH, EMB), jnp.float32),
            pltpu.VMEM((CH, EMB), jnp.float32),
            pltpu.VMEM_SHARED((N, EMB), jnp.float32),
            pltpu.SemaphoreType.DMA((2,)),
            pltpu.SemaphoreType.DMA((NSLOT,)),
            pltpu.SemaphoreType.DMA((NSLOT,)),
        ],
    )(_sc_body)


def kernel(x, degree, W_in, b_in, W1, b1, W2, b2, ln_s, ln_b,
           out_ln_s, out_ln_b, W_out, b_out, edge_index):
    deg = degree.reshape(N, 1)
    src = edge_index[0].reshape(NW, SEG, CPS, CH)
    dst = edge_index[1].reshape(NW, SEG, CPS, CH)
    zeros = jnp.zeros((RPB, EMB), jnp.float32)

    h, p = _in_msg_call(x, W_in, b_in.reshape(1, EMB), deg,
                        W1[0], b1[0].reshape(1, HID))
    for l in range(L - 1):
        q = _make_sc_call()(p, src, dst, zeros)
        h, p = _upd_msg_call(q[0], q[1], p, h, deg, W2[l],
                             b2[l].reshape(1, EMB), ln_s[l].reshape(1, EMB),
                             ln_b[l].reshape(1, EMB), W1[l + 1],
                             b1[l + 1].reshape(1, HID))
    q = _make_sc_call()(p, src, dst, zeros)
    return _upd_out_call(q[0], q[1], p, h, deg, W2[L - 1],
                         b2[L - 1].reshape(1, EMB),
                         ln_s[L - 1].reshape(1, EMB),
                         ln_b[L - 1].reshape(1, EMB),
                         out_ln_s.reshape(1, EMB), out_ln_b.reshape(1, EMB),
                         W_out, b_out.reshape(1, OUT))


# speedup vs baseline: 1.2232x; 1.0345x over previous
import functools

import jax
import jax.numpy as jnp
from jax import lax
from jax.experimental import pallas as pl
from jax.experimental.pallas import tpu as pltpu
from jax.experimental.pallas import tpu_sc as plsc

N = 10000
E = 320000
D_IN = 128
EMB = 128
HID = 128
OUT = 64
L = 3

NC = 2
NS = 16
NW = NC * NS
EPW = E // NW
CH = 80
NCHUNK = EPW // CH
SEG = 5
CPS = NCHUNK // SEG
NSLOT = 3
RPB = 624
TAIL = N - NS * RPB

ROWS_B = 2000
GRID = N // ROWS_B


def _ln_rows(t, s, b):
    mu = jnp.mean(t, axis=-1, keepdims=True)
    var = jnp.mean((t - mu) ** 2, axis=-1, keepdims=True)
    return (t - mu) * lax.rsqrt(var + 1e-5) * s + b


def _in_msg_body(x_ref, win_ref, bin_ref, deg_ref, w1_ref, b1_ref,
                 h_ref, p_ref):
    h = jax.nn.relu(
        jnp.dot(x_ref[...], win_ref[...], preferred_element_type=jnp.float32)
        + bin_ref[...]
    )
    h_ref[...] = h
    inv = lax.rsqrt(jnp.maximum(deg_ref[...] + 1.0, 1.0))
    m = jax.nn.relu(
        jnp.dot(h, w1_ref[...], preferred_element_type=jnp.float32)
        + b1_ref[...]
    )
    p_ref[...] = m * inv


def _upd_msg_body(q0_ref, q1_ref, p_ref, h_ref, deg_ref, w_ref, b_ref,
                  s_ref, lb_ref, w1_ref, b1_ref, h_ref_o, p_ref_o):
    inv = lax.rsqrt(jnp.maximum(deg_ref[...] + 1.0, 1.0))
    agg = (q0_ref[...] + q1_ref[...] + p_ref[...]) * inv
    t = jnp.dot(agg, w_ref[...], preferred_element_type=jnp.float32) + b_ref[...]
    hn = _ln_rows(t, s_ref[...], lb_ref[...]) + h_ref[...]
    h_ref_o[...] = hn
    m = jax.nn.relu(
        jnp.dot(hn, w1_ref[...], preferred_element_type=jnp.float32)
        + b1_ref[...]
    )
    p_ref_o[...] = m * inv


def _upd_out_body(q0_ref, q1_ref, p_ref, h_ref, deg_ref, w_ref, b_ref,
                  s_ref, lb_ref, os_ref, ob_ref, wo_ref, bo_ref, o_ref):
    inv = lax.rsqrt(jnp.maximum(deg_ref[...] + 1.0, 1.0))
    agg = (q0_ref[...] + q1_ref[...] + p_ref[...]) * inv
    t = jnp.dot(agg, w_ref[...], preferred_element_type=jnp.float32) + b_ref[...]
    hn = _ln_rows(t, s_ref[...], lb_ref[...]) + h_ref[...]
    t2 = _ln_rows(hn, os_ref[...], ob_ref[...])
    o_ref[...] = (
        jnp.dot(t2, wo_ref[...], preferred_element_type=jnp.float32)
        + bo_ref[...]
    )


def _row_spec(cols):
    return pl.BlockSpec((ROWS_B, cols), lambda i: (i, 0))


def _rep_spec(r, c):
    return pl.BlockSpec((r, c), lambda i: (0, 0))


_in_msg_call = pl.pallas_call(
    _in_msg_body,
    grid=(GRID,),
    in_specs=[
        _row_spec(D_IN),
        _rep_spec(D_IN, EMB),
        _rep_spec(1, EMB),
        _row_spec(1),
        _rep_spec(EMB, HID),
        _rep_spec(1, HID),
    ],
    out_specs=[_row_spec(EMB), _row_spec(HID)],
    out_shape=[
        jax.ShapeDtypeStruct((N, EMB), jnp.float32),
        jax.ShapeDtypeStruct((N, HID), jnp.float32),
    ],
)

_upd_msg_call = pl.pallas_call(
    _upd_msg_body,
    grid=(GRID,),
    in_specs=[
        _row_spec(HID),
        _row_spec(HID),
        _row_spec(HID),
        _row_spec(EMB),
        _row_spec(1),
        _rep_spec(HID, EMB),
        _rep_spec(1, EMB),
        _rep_spec(1, EMB),
        _rep_spec(1, EMB),
        _rep_spec(EMB, HID),
        _rep_spec(1, HID),
    ],
    out_specs=[_row_spec(EMB), _row_spec(HID)],
    out_shape=[
        jax.ShapeDtypeStruct((N, EMB), jnp.float32),
        jax.ShapeDtypeStruct((N, HID), jnp.float32),
    ],
)

_upd_out_call = pl.pallas_call(
    _upd_out_body,
    grid=(GRID,),
    in_specs=[
        _row_spec(HID),
        _row_spec(HID),
        _row_spec(HID),
        _row_spec(EMB),
        _row_spec(1),
        _rep_spec(HID, EMB),
        _rep_spec(1, EMB),
        _rep_spec(1, EMB),
        _rep_spec(1, EMB),
        _rep_spec(1, EMB),
        _rep_spec(1, EMB),
        _rep_spec(EMB, OUT),
        _rep_spec(1, OUT),
    ],
    out_specs=_row_spec(OUT),
    out_shape=jax.ShapeDtypeStruct((N, OUT), jnp.float32),
)


def _sc_body(p_hbm, src_hbm, dst_hbm, zeros_hbm, out_hbm,
             src_a, dst_a, src_b, dst_b, rows0, rows1, rows2, acc,
             lsem, gsem, ssem):
    rows = (rows0, rows1, rows2)
    slabs = ((src_a, dst_a), (src_b, dst_b))
    c = lax.axis_index("c")
    s = lax.axis_index("s")
    wid = c * NS + s

    def load_slab(seg, t):
        pltpu.async_copy(src_hbm.at[wid, seg], slabs[t][0], lsem.at[t])
        pltpu.async_copy(dst_hbm.at[wid, seg], slabs[t][1], lsem.at[t])

    def wait_slab(t):
        pltpu.make_async_copy(src_hbm.at[wid, 0], slabs[t][0],
                              lsem.at[t]).wait()
        pltpu.make_async_copy(dst_hbm.at[wid, 0], slabs[t][1],
                              lsem.at[t]).wait()

    load_slab(0, 0)
    load_slab(1, 1)
    pltpu.sync_copy(zeros_hbm.at[pl.ds(0, RPB)], acc.at[pl.ds(s * RPB, RPB)])

    @pl.when(s == 0)
    def _zero_tail():
        pltpu.sync_copy(zeros_hbm.at[pl.ds(0, TAIL)],
                        acc.at[pl.ds(NS * RPB, TAIL)])

    plsc.subcore_barrier()

    def issue_gather(sv, cc, b):
        pltpu.async_copy(p_hbm.at[sv.at[cc]], rows[b], gsem.at[b])

    def wait_gather(sv, cc, b):
        pltpu.make_async_copy(p_hbm.at[sv.at[cc]], rows[b],
                              gsem.at[b]).wait()

    def issue_scatter(dv, cc, b):
        pltpu.async_copy(rows[b], acc.at[dv.at[cc]], ssem.at[b], add=True)

    def wait_scatter(dv, cc, b):
        pltpu.make_async_copy(rows[b], acc.at[dv.at[cc]],
                              ssem.at[b]).wait()

    def step(sv, dv, cc, b, do_swait, do_gather):
        wait_gather(sv, cc, b)
        if do_swait:
            wait_scatter(dv, cc - 1, (b + 2) % NSLOT)
        if do_gather:
            issue_gather(sv, cc + 2, (b + 2) % NSLOT)
        issue_scatter(dv, cc, b)

    for seg in range(SEG):
        t = seg % 2
        sv, dv = slabs[t]
        wait_slab(t)
        issue_gather(sv, 0, 0)
        issue_gather(sv, 1, 1)
        step(sv, dv, 0, 0, do_swait=False, do_gather=True)
        step(sv, dv, 1, 1, do_swait=True, do_gather=True)

        def trip(j, carry, sv=sv, dv=dv):
            cb = NSLOT * j + 2
            step(sv, dv, cb, 2, True, True)
            step(sv, dv, cb + 1, 0, True, True)
            step(sv, dv, cb + 2, 1, True, True)
            return carry

        lax.fori_loop(0, (CPS - 4) // NSLOT, trip, 0)
        step(sv, dv, CPS - 2, (CPS - 2) % NSLOT, True, False)
        step(sv, dv, CPS - 1, (CPS - 1) % NSLOT, True, False)
        wait_scatter(dv, CPS - 1, (CPS - 1) % NSLOT)
        if seg + 2 < SEG:
            load_slab(seg + 2, t)

    plsc.subcore_barrier()
    pltpu.sync_copy(acc.at[pl.ds(s * RPB, RPB)],
                    out_hbm.at[c].at[pl.ds(s * RPB, RPB)])

    @pl.when(s == 0)
    def _drain_tail():
        pltpu.sync_copy(acc.at[pl.ds(NS * RPB, TAIL)],
                        out_hbm.at[c].at[pl.ds(NS * RPB, TAIL)])


@functools.lru_cache(maxsize=None)
def _make_sc_call():
    return functools.partial(
        pl.kernel,
        out_type=jax.ShapeDtypeStruct((NC, N, EMB), jnp.float32),
        mesh=plsc.VectorSubcoreMesh(core_axis_name="c", subcore_axis_name="s",
                                    num_cores=NC, num_subcores=NS),
        scratch_types=[
            pltpu.VMEM((CPS, CH), jnp.int32),
            pltpu.VMEM((CPS, CH), jnp.int32),
            pltpu.VMEM((CPS, CH), jnp.int32),
            pltpu.VMEM((CPS, CH), jnp.int32),
            pltpu.VMEM((CH, EMB), jnp.float32),
            pltpu.VMEM((CH, EMB), jnp.float32),
            pltpu.VMEM((CH, EMB), jnp.float32),
            pltpu.VMEM_SHARED((N, EMB), jnp.float32),
            pltpu.SemaphoreType.DMA((2,)),
            pltpu.SemaphoreType.DMA((NSLOT,)),
            pltpu.SemaphoreType.DMA((NSLOT,)),
        ],
    )(_sc_body)


def kernel(x, degree, W_in, b_in, W1, b1, W2, b2, ln_s, ln_b,
           out_ln_s, out_ln_b, W_out, b_out, edge_index):
    deg = degree.reshape(N, 1)
    src = edge_index[0].reshape(NW, SEG, CPS, CH)
    dst = edge_index[1].reshape(NW, SEG, CPS, CH)
    zeros = jnp.zeros((RPB, EMB), jnp.float32)

    h, p = _in_msg_call(x, W_in, b_in.reshape(1, EMB), deg,
                        W1[0], b1[0].reshape(1, HID))
    for l in range(L - 1):
        q = _make_sc_call()(p, src, dst, zeros)
        h, p = _upd_msg_call(q[0], q[1], p, h, deg, W2[l],
                             b2[l].reshape(1, EMB), ln_s[l].reshape(1, EMB),
                             ln_b[l].reshape(1, EMB), W1[l + 1],
                             b1[l + 1].reshape(1, HID))
    q = _make_sc_call()(p, src, dst, zeros)
    return _upd_out_call(q[0], q[1], p, h, deg, W2[L - 1],
                         b2[L - 1].reshape(1, EMB),
                         ln_s[L - 1].reshape(1, EMB),
                         ln_b[L - 1].reshape(1, EMB),
                         out_ln_s.reshape(1, EMB), out_ln_b.reshape(1, EMB),
                         W_out, b_out.reshape(1, OUT))


# speedup vs baseline: 1.2355x; 1.0101x over previous
import functools

import jax
import jax.numpy as jnp
from jax import lax
from jax.experimental import pallas as pl
from jax.experimental.pallas import tpu as pltpu
from jax.experimental.pallas import tpu_sc as plsc

N = 10000
E = 320000
D_IN = 128
EMB = 128
HID = 128
OUT = 64
L = 3

NC = 2
NS = 16
NW = NC * NS
EPW = E // NW
CH = 80
NCHUNK = EPW // CH
SEG = 5
CPS = NCHUNK // SEG
NSLOT = 3
RPB = 624
TAIL = N - NS * RPB

ROWS_B = 5000
GRID = N // ROWS_B


def _ln_rows(t, s, b):
    mu = jnp.mean(t, axis=-1, keepdims=True)
    var = jnp.mean((t - mu) ** 2, axis=-1, keepdims=True)
    return (t - mu) * lax.rsqrt(var + 1e-5) * s + b


def _in_msg_body(x_ref, win_ref, bin_ref, deg_ref, w1_ref, b1_ref,
                 h_ref, p_ref):
    h = jax.nn.relu(
        jnp.dot(x_ref[...], win_ref[...], preferred_element_type=jnp.float32)
        + bin_ref[...]
    )
    h_ref[...] = h
    inv = lax.rsqrt(jnp.maximum(deg_ref[...] + 1.0, 1.0))
    m = jax.nn.relu(
        jnp.dot(h, w1_ref[...], preferred_element_type=jnp.float32)
        + b1_ref[...]
    )
    p_ref[...] = m * inv


def _upd_msg_body(q0_ref, q1_ref, p_ref, h_ref, deg_ref, w_ref, b_ref,
                  s_ref, lb_ref, w1_ref, b1_ref, h_ref_o, p_ref_o):
    inv = lax.rsqrt(jnp.maximum(deg_ref[...] + 1.0, 1.0))
    agg = (q0_ref[...] + q1_ref[...] + p_ref[...]) * inv
    t = jnp.dot(agg, w_ref[...], preferred_element_type=jnp.float32) + b_ref[...]
    hn = _ln_rows(t, s_ref[...], lb_ref[...]) + h_ref[...]
    h_ref_o[...] = hn
    m = jax.nn.relu(
        jnp.dot(hn, w1_ref[...], preferred_element_type=jnp.float32)
        + b1_ref[...]
    )
    p_ref_o[...] = m * inv


def _upd_out_body(q0_ref, q1_ref, p_ref, h_ref, deg_ref, w_ref, b_ref,
                  s_ref, lb_ref, os_ref, ob_ref, wo_ref, bo_ref, o_ref):
    inv = lax.rsqrt(jnp.maximum(deg_ref[...] + 1.0, 1.0))
    agg = (q0_ref[...] + q1_ref[...] + p_ref[...]) * inv
    t = jnp.dot(agg, w_ref[...], preferred_element_type=jnp.float32) + b_ref[...]
    hn = _ln_rows(t, s_ref[...], lb_ref[...]) + h_ref[...]
    t2 = _ln_rows(hn, os_ref[...], ob_ref[...])
    o_ref[...] = (
        jnp.dot(t2, wo_ref[...], preferred_element_type=jnp.float32)
        + bo_ref[...]
    )


def _row_spec(cols):
    return pl.BlockSpec((ROWS_B, cols), lambda i: (i, 0))


def _rep_spec(r, c):
    return pl.BlockSpec((r, c), lambda i: (0, 0))


_in_msg_call = pl.pallas_call(
    _in_msg_body,
    grid=(GRID,),
    in_specs=[
        _row_spec(D_IN),
        _rep_spec(D_IN, EMB),
        _rep_spec(1, EMB),
        _row_spec(1),
        _rep_spec(EMB, HID),
        _rep_spec(1, HID),
    ],
    out_specs=[_row_spec(EMB), _row_spec(HID)],
    out_shape=[
        jax.ShapeDtypeStruct((N, EMB), jnp.float32),
        jax.ShapeDtypeStruct((N, HID), jnp.float32),
    ],
)

_upd_msg_call = pl.pallas_call(
    _upd_msg_body,
    grid=(GRID,),
    in_specs=[
        _row_spec(HID),
        _row_spec(HID),
        _row_spec(HID),
        _row_spec(EMB),
        _row_spec(1),
        _rep_spec(HID, EMB),
        _rep_spec(1, EMB),
        _rep_spec(1, EMB),
        _rep_spec(1, EMB),
        _rep_spec(EMB, HID),
        _rep_spec(1, HID),
    ],
    out_specs=[_row_spec(EMB), _row_spec(HID)],
    out_shape=[
        jax.ShapeDtypeStruct((N, EMB), jnp.float32),
        jax.ShapeDtypeStruct((N, HID), jnp.float32),
    ],
)

_upd_out_call = pl.pallas_call(
    _upd_out_body,
    grid=(GRID,),
    in_specs=[
        _row_spec(HID),
        _row_spec(HID),
        _row_spec(HID),
        _row_spec(EMB),
        _row_spec(1),
        _rep_spec(HID, EMB),
        _rep_spec(1, EMB),
        _rep_spec(1, EMB),
        _rep_spec(1, EMB),
        _rep_spec(1, EMB),
        _rep_spec(1, EMB),
        _rep_spec(EMB, OUT),
        _rep_spec(1, OUT),
    ],
    out_specs=_row_spec(OUT),
    out_shape=jax.ShapeDtypeStruct((N, OUT), jnp.float32),
)


def _sc_body(p_hbm, src_hbm, dst_hbm, zeros_hbm, out_hbm,
             src_a, dst_a, src_b, dst_b, rows0, rows1, rows2, acc,
             lsem, gsem, ssem):
    rows = (rows0, rows1, rows2)
    slabs = ((src_a, dst_a), (src_b, dst_b))
    c = lax.axis_index("c")
    s = lax.axis_index("s")
    wid = c * NS + s

    def load_slab(seg, t):
        pltpu.async_copy(src_hbm.at[wid, seg], slabs[t][0], lsem.at[t])
        pltpu.async_copy(dst_hbm.at[wid, seg], slabs[t][1], lsem.at[t])

    def wait_slab(t):
        pltpu.make_async_copy(src_hbm.at[wid, 0], slabs[t][0],
                              lsem.at[t]).wait()
        pltpu.make_async_copy(dst_hbm.at[wid, 0], slabs[t][1],
                              lsem.at[t]).wait()

    load_slab(0, 0)
    load_slab(1, 1)
    pltpu.sync_copy(zeros_hbm.at[pl.ds(0, RPB)], acc.at[pl.ds(s * RPB, RPB)])

    @pl.when(s == 0)
    def _zero_tail():
        pltpu.sync_copy(zeros_hbm.at[pl.ds(0, TAIL)],
                        acc.at[pl.ds(NS * RPB, TAIL)])

    plsc.subcore_barrier()

    def issue_gather(sv, cc, b):
        pltpu.async_copy(p_hbm.at[sv.at[cc]], rows[b], gsem.at[b])

    def wait_gather(sv, cc, b):
        pltpu.make_async_copy(p_hbm.at[sv.at[cc]], rows[b],
                              gsem.at[b]).wait()

    def issue_scatter(dv, cc, b):
        pltpu.async_copy(rows[b], acc.at[dv.at[cc]], ssem.at[b], add=True)

    def wait_scatter(dv, cc, b):
        pltpu.make_async_copy(rows[b], acc.at[dv.at[cc]],
                              ssem.at[b]).wait()

    def step(sv, dv, cc, b, do_swait, do_gather):
        wait_gather(sv, cc, b)
        if do_swait:
            wait_scatter(dv, cc - 1, (b + 2) % NSLOT)
        if do_gather:
            issue_gather(sv, cc + 2, (b + 2) % NSLOT)
        issue_scatter(dv, cc, b)

    for seg in range(SEG):
        t = seg % 2
        sv, dv = slabs[t]
        wait_slab(t)
        issue_gather(sv, 0, 0)
        issue_gather(sv, 1, 1)
        step(sv, dv, 0, 0, do_swait=False, do_gather=True)
        step(sv, dv, 1, 1, do_swait=True, do_gather=True)

        def trip(j, carry, sv=sv, dv=dv):
            cb = NSLOT * j + 2
            step(sv, dv, cb, 2, True, True)
            step(sv, dv, cb + 1, 0, True, True)
            step(sv, dv, cb + 2, 1, True, True)
            return carry

        lax.fori_loop(0, (CPS - 4) // NSLOT, trip, 0)
        step(sv, dv, CPS - 2, (CPS - 2) % NSLOT, True, False)
        step(sv, dv, CPS - 1, (CPS - 1) % NSLOT, True, False)
        wait_scatter(dv, CPS - 1, (CPS - 1) % NSLOT)
        if seg + 2 < SEG:
            load_slab(seg + 2, t)

    plsc.subcore_barrier()
    pltpu.sync_copy(acc.at[pl.ds(s * RPB, RPB)],
                    out_hbm.at[c].at[pl.ds(s * RPB, RPB)])

    @pl.when(s == 0)
    def _drain_tail():
        pltpu.sync_copy(acc.at[pl.ds(NS * RPB, TAIL)],
                        out_hbm.at[c].at[pl.ds(NS * RPB, TAIL)])


@functools.lru_cache(maxsize=None)
def _make_sc_call():
    return functools.partial(
        pl.kernel,
        out_type=jax.ShapeDtypeStruct((NC, N, EMB), jnp.float32),
        mesh=plsc.VectorSubcoreMesh(core_axis_name="c", subcore_axis_name="s",
                                    num_cores=NC, num_subcores=NS),
        scratch_types=[
            pltpu.VMEM((CPS, CH), jnp.int32),
            pltpu.VMEM((CPS, CH), jnp.int32),
            pltpu.VMEM((CPS, CH), jnp.int32),
            pltpu.VMEM((CPS, CH), jnp.int32),
            pltpu.VMEM((CH, EMB), jnp.float32),
            pltpu.VMEM((CH, EMB), jnp.float32),
            pltpu.VMEM((CH, EMB), jnp.float32),
            pltpu.VMEM_SHARED((N, EMB), jnp.float32),
            pltpu.SemaphoreType.DMA((2,)),
            pltpu.SemaphoreType.DMA((NSLOT,)),
            pltpu.SemaphoreType.DMA((NSLOT,)),
        ],
    )(_sc_body)


def kernel(x, degree, W_in, b_in, W1, b1, W2, b2, ln_s, ln_b,
           out_ln_s, out_ln_b, W_out, b_out, edge_index):
    deg = degree.reshape(N, 1)
    src = edge_index[0].reshape(NW, SEG, CPS, CH)
    dst = edge_index[1].reshape(NW, SEG, CPS, CH)
    zeros = jnp.zeros((RPB, EMB), jnp.float32)

    h, p = _in_msg_call(x, W_in, b_in.reshape(1, EMB), deg,
                        W1[0], b1[0].reshape(1, HID))
    for l in range(L - 1):
        q = _make_sc_call()(p, src, dst, zeros)
        h, p = _upd_msg_call(q[0], q[1], p, h, deg, W2[l],
                             b2[l].reshape(1, EMB), ln_s[l].reshape(1, EMB),
                             ln_b[l].reshape(1, EMB), W1[l + 1],
                             b1[l + 1].reshape(1, HID))
    q = _make_sc_call()(p, src, dst, zeros)
    return _upd_out_call(q[0], q[1], p, h, deg, W2[L - 1],
                         b2[L - 1].reshape(1, EMB),
                         ln_s[L - 1].reshape(1, EMB),
                         ln_b[L - 1].reshape(1, EMB),
                         out_ln_s.reshape(1, EMB), out_ln_b.reshape(1, EMB),
                         W_out, b_out.reshape(1, OUT))
